# baseline (speedup 1.0000x reference)
import functools

import jax
import jax.numpy as jnp
from jax import lax
from jax.experimental import pallas as pl
from jax.experimental.pallas import tpu as pltpu
from jax.experimental.pallas import tpu_sc as plsc

B, T, C = 4, 2048, 1024


def kernel(x, table):
    del x
    info = plsc.get_sparse_core_info()
    nw = info.num_cores * info.num_subcores
    rows = T // nw
    nch = 4
    ch = rows // nch
    mesh = plsc.VectorSubcoreMesh(core_axis_name="c", subcore_axis_name="s")

    @functools.partial(
        pl.kernel,
        mesh=mesh,
        out_type=jax.ShapeDtypeStruct((B, T, C), jnp.float32),
        scratch_types=[
            pltpu.VMEM((nch, ch, C), jnp.float32),
            pltpu.SemaphoreType.DMA,
            pltpu.SemaphoreType.DMA,
        ],
    )
    def body(table_hbm, out_hbm, buf, rsem, wsem):
        wid = lax.axis_index("s") * info.num_cores + lax.axis_index("c")
        base = wid * rows
        reads = [
            pltpu.async_copy(
                table_hbm.at[pl.ds(base + i * ch, ch)], buf.at[i], rsem
            )
            for i in range(nch)
        ]
        writes = []
        for i in range(nch):
            reads[i].wait()
            writes += [
                pltpu.async_copy(
                    buf.at[i], out_hbm.at[b, pl.ds(base + i * ch, ch)], wsem
                )
                for b in range(B)
            ]
        for c in writes:
            c.wait()

    return body(table)


# speedup vs baseline: 1.0180x; 1.0180x over previous
import functools

import jax
import jax.numpy as jnp
from jax import lax
from jax.experimental import pallas as pl
from jax.experimental.pallas import tpu as pltpu
from jax.experimental.pallas import tpu_sc as plsc

B, T, C = 4, 2048, 1024


def kernel(x, table):
    del x
    info = plsc.get_sparse_core_info()
    nw = info.num_cores * info.num_subcores
    rows = T // nw
    ch = rows // 2
    mesh = plsc.VectorSubcoreMesh(core_axis_name="c", subcore_axis_name="s")

    @functools.partial(
        pl.kernel,
        mesh=mesh,
        out_type=jax.ShapeDtypeStruct((B, T, C), jnp.float32),
        scratch_types=[
            pltpu.VMEM((2, ch, C), jnp.float32),
            pltpu.SemaphoreType.DMA,
            pltpu.SemaphoreType.DMA,
        ],
    )
    def body(table_hbm, out_hbm, buf, rsem, wsem):
        wid = lax.axis_index("s") * info.num_cores + lax.axis_index("c")
        base = wid * rows
        r0 = pltpu.async_copy(table_hbm.at[pl.ds(base, ch)], buf.at[0], rsem)
        r1 = pltpu.async_copy(table_hbm.at[pl.ds(base + ch, ch)], buf.at[1], rsem)
        r0.wait()
        w0 = [
            pltpu.async_copy(buf.at[0], out_hbm.at[b, pl.ds(base, ch)], wsem)
            for b in range(B)
        ]
        r1.wait()
        w1 = [
            pltpu.async_copy(buf.at[1], out_hbm.at[b, pl.ds(base + ch, ch)], wsem)
            for b in range(B)
        ]
        for c in w0 + w1:
            c.wait()

    return body(table)
